# baseline (device time: 80515 ns/iter reference)
import jax
import jax.numpy as jnp
from jax import lax
from jax.experimental import pallas as pl
from jax.experimental.pallas import tpu as pltpu

N_DEV = 8
SQ = 1024
SKV_LOCAL = 1024
HQ = 8
DH = 128
D = HQ * DH
SCALE = 0.08838834764831843
BLK = 64
NS = 2
RS = SQ // NS


def _body(x_ref, wq_ref, k_ref, v_ref, wo_ref, out_ref,
          q3_ref, k3_ref, v3_ref, stage_ref, lstage_ref,
          comm_ref, lcomm_ref, ctx_ref,
          send1, recv1, sendl, recvl, send2, recv2):
    pos = lax.axis_index("i")

    barrier_sem = pltpu.get_barrier_semaphore()
    for off in range(1, N_DEV):
        pl.semaphore_signal(barrier_sem, inc=1,
                            device_id=(lax.rem(pos + off, N_DEV),),
                            device_id_type=pl.DeviceIdType.MESH)
    pl.semaphore_wait(barrier_sem, N_DEV - 1)

    q = jnp.dot(x_ref[...].astype(jnp.bfloat16),
                wq_ref[...].astype(jnp.bfloat16),
                preferred_element_type=jnp.float32).astype(jnp.bfloat16)
    for h in range(HQ):
        sl = slice(h * DH, (h + 1) * DH)
        q3_ref[h] = q[:, sl]
        k3_ref[h] = k_ref[:, sl].astype(jnp.bfloat16)
        v3_ref[h] = v_ref[:, sl].astype(jnp.bfloat16)

    qb = lax.broadcasted_iota(jnp.int32, (SQ, SKV_LOCAL), 0) // BLK
    kb = (lax.broadcasted_iota(jnp.int32, (SQ, SKV_LOCAL), 1)
          + pos * SKV_LOCAL) // BLK
    mask = (qb == kb) | (kb == 0) | ((qb + kb) % 3 == 0)
    ones_col = jnp.ones((SKV_LOCAL, 1), jnp.bfloat16)

    rows = [slice(s * RS, (s + 1) * RS) for s in range(NS)]

    def p1_send(s, idx, hd):
        return pltpu.make_async_remote_copy(
            src_ref=stage_ref.at[hd, rows[s]],
            dst_ref=comm_ref.at[pos, rows[s]],
            send_sem=send1.at[idx + N_DEV * s],
            recv_sem=recv1.at[pos + N_DEV * s],
            device_id=(hd,),
            device_id_type=pl.DeviceIdType.MESH,
        )

    def l_send(s, off):
        return pltpu.make_async_remote_copy(
            src_ref=lstage_ref.at[s],
            dst_ref=lcomm_ref.at[pos, rows[s]],
            send_sem=sendl.at[off + N_DEV * s],
            recv_sem=recvl.at[pos + N_DEV * s],
            device_id=(lax.rem(pos + off, N_DEV),),
            device_id_type=pl.DeviceIdType.MESH,
        )

    def p2_send(s, off):
        return pltpu.make_async_remote_copy(
            src_ref=stage_ref.at[pos, rows[s]],
            dst_ref=ctx_ref.at[pos, rows[s]],
            send_sem=send2.at[off + N_DEV * s],
            recv_sem=recv2.at[pos + N_DEV * s],
            device_id=(lax.rem(pos + off, N_DEV),),
            device_id_type=pl.DeviceIdType.MESH,
        )

    def head_partial(s, hd):
        sc = lax.dot_general(q3_ref[hd, rows[s]], k3_ref[hd],
                             (((1,), (1,)), ((), ())),
                             preferred_element_type=jnp.float32) * SCALE
        p = jnp.where(mask[rows[s]], jnp.exp(sc), 0.0).astype(jnp.bfloat16)
        o_h = jnp.dot(p, v3_ref[hd],
                      preferred_element_type=jnp.float32)
        l_h = jnp.dot(p, ones_col,
                      preferred_element_type=jnp.float32)
        return o_h, l_h

    ctx_mine = [None] * NS
    for s in range(NS):
        l_parts = []
        for idx in range(N_DEV - 1):
            hd = lax.rem(pos + 1 + idx, N_DEV)
            o_h, l_h = head_partial(s, hd)
            l_parts.append(l_h)
            stage_ref[hd, rows[s]] = o_h.astype(jnp.bfloat16)
            p1_send(s, idx, hd).start()
        o_own, l_own = head_partial(s, pos)

        lstage_ref[s] = jnp.concatenate(
            l_parts + [l_own], axis=1).astype(jnp.bfloat16)
        for off in range(1, N_DEV):
            l_send(s, off).start()

        acc = o_own
        l_col = l_own
        for k in range(N_DEV - 1):
            src = lax.rem(pos + (N_DEV - 1) - k, N_DEV)
            pltpu.make_async_remote_copy(
                src_ref=comm_ref.at[src, rows[s]],
                dst_ref=comm_ref.at[src, rows[s]],
                send_sem=send1.at[0],
                recv_sem=recv1.at[src + N_DEV * s],
                device_id=(pos,),
                device_id_type=pl.DeviceIdType.MESH,
            ).wait_recv()
            acc = acc + comm_ref[src, rows[s]].astype(jnp.float32)
        for k in range(N_DEV - 1):
            src = lax.rem(pos + (N_DEV - 1) - k, N_DEV)
            pltpu.make_async_remote_copy(
                src_ref=lcomm_ref.at[src, rows[s]],
                dst_ref=lcomm_ref.at[src, rows[s]],
                send_sem=sendl.at[0],
                recv_sem=recvl.at[src + N_DEV * s],
                device_id=(pos,),
                device_id_type=pl.DeviceIdType.MESH,
            ).wait_recv()
            l_col = l_col + lcomm_ref[src, rows[s], k:k + 1].astype(
                jnp.float32)

        ctx_mine[s] = (acc / l_col).astype(jnp.bfloat16)
        stage_ref[pos, rows[s]] = ctx_mine[s]
        for off in range(1, N_DEV):
            p2_send(s, off).start()

    for s in range(NS):
        out = jnp.dot(ctx_mine[s],
                      wo_ref[pl.ds(pos * DH, DH), :].astype(jnp.bfloat16),
                      preferred_element_type=jnp.float32)
        for k in range(N_DEV - 1):
            src = lax.rem(pos + (N_DEV - 1) - k, N_DEV)
            pltpu.make_async_remote_copy(
                src_ref=ctx_ref.at[src, rows[s]],
                dst_ref=ctx_ref.at[src, rows[s]],
                send_sem=send2.at[0],
                recv_sem=recv2.at[src + N_DEV * s],
                device_id=(pos,),
                device_id_type=pl.DeviceIdType.MESH,
            ).wait_recv()
            out = out + jnp.dot(
                ctx_ref[src, rows[s]],
                wo_ref[pl.ds(src * DH, DH), :].astype(jnp.bfloat16),
                preferred_element_type=jnp.float32)
        out_ref[0, rows[s]] = out

    for s in range(NS):
        for idx in range(N_DEV - 1):
            p1_send(s, idx, lax.rem(pos + 1 + idx, N_DEV)).wait_send()
        for off in range(1, N_DEV):
            l_send(s, off).wait_send()
            p2_send(s, off).wait_send()


def kernel(x, Wq, K_ext, V_ext, Wo):
    return pl.pallas_call(
        _body,
        out_shape=jax.ShapeDtypeStruct((1, SQ, D), jnp.float32),
        in_specs=[pl.BlockSpec(memory_space=pltpu.VMEM)] * 5,
        out_specs=pl.BlockSpec(memory_space=pltpu.VMEM),
        scratch_shapes=[
            pltpu.VMEM((HQ, SQ, DH), jnp.bfloat16),
            pltpu.VMEM((HQ, SKV_LOCAL, DH), jnp.bfloat16),
            pltpu.VMEM((HQ, SKV_LOCAL, DH), jnp.bfloat16),
            pltpu.VMEM((HQ, SQ, DH), jnp.bfloat16),
            pltpu.VMEM((NS, RS, HQ), jnp.bfloat16),
            pltpu.VMEM((N_DEV, SQ, DH), jnp.bfloat16),
            pltpu.VMEM((N_DEV, SQ, HQ), jnp.bfloat16),
            pltpu.VMEM((N_DEV, SQ, DH), jnp.bfloat16),
            pltpu.SemaphoreType.DMA((NS * N_DEV,)),
            pltpu.SemaphoreType.DMA((NS * N_DEV,)),
            pltpu.SemaphoreType.DMA((NS * N_DEV,)),
            pltpu.SemaphoreType.DMA((NS * N_DEV,)),
            pltpu.SemaphoreType.DMA((NS * N_DEV,)),
            pltpu.SemaphoreType.DMA((NS * N_DEV,)),
        ],
        compiler_params=pltpu.CompilerParams(
            collective_id=0,
            vmem_limit_bytes=100 * 1024 * 1024,
        ),
    )(x[0], Wq, K_ext[0].reshape(SKV_LOCAL, D), V_ext[0].reshape(SKV_LOCAL, D),
      Wo)


# device time: 69976 ns/iter; 1.1506x vs baseline; 1.1506x over previous
import jax
import jax.numpy as jnp
from jax import lax
from jax.experimental import pallas as pl
from jax.experimental.pallas import tpu as pltpu

N_DEV = 8
SQ = 1024
SKV_LOCAL = 1024
HQ = 8
DH = 128
D = HQ * DH
SCALE = 0.08838834764831843
BLK = 64
PACK = SQ + HQ


def _body(x_ref, wq_ref, k_ref, v_ref, wo_ref, out_ref,
          q3_ref, k3_ref, v3_ref, stage_ref, comm_ref, ctx_ref,
          send1, recv1, send2, recv2):
    pos = lax.axis_index("i")

    barrier_sem = pltpu.get_barrier_semaphore()
    for off in range(1, N_DEV):
        pl.semaphore_signal(barrier_sem, inc=1,
                            device_id=(lax.rem(pos + off, N_DEV),),
                            device_id_type=pl.DeviceIdType.MESH)
    pl.semaphore_wait(barrier_sem, N_DEV - 1)

    q = jnp.dot(x_ref[...].astype(jnp.bfloat16),
                wq_ref[...].astype(jnp.bfloat16),
                preferred_element_type=jnp.float32).astype(jnp.bfloat16)
    for h in range(HQ):
        sl = slice(h * DH, (h + 1) * DH)
        q3_ref[h] = q[:, sl]
        k3_ref[h] = k_ref[:, sl].astype(jnp.bfloat16)
        v3_ref[h] = v_ref[:, sl].astype(jnp.bfloat16)

    qb = lax.broadcasted_iota(jnp.int32, (SQ, SKV_LOCAL), 0) // BLK
    kb = (lax.broadcasted_iota(jnp.int32, (SQ, SKV_LOCAL), 1)
          + pos * SKV_LOCAL) // BLK
    mask = (qb == kb) | (kb == 0) | ((qb + kb) % 3 == 0)
    ones_row = jnp.ones((1, SKV_LOCAL), jnp.bfloat16)
    ieye = (lax.broadcasted_iota(jnp.int32, (DH, DH), 0)
            == lax.broadcasted_iota(jnp.int32, (DH, DH), 1)
            ).astype(jnp.bfloat16)

    def head_partial(hd):
        sc = lax.dot_general(q3_ref[hd], k3_ref[hd],
                             (((1,), (1,)), ((), ())),
                             preferred_element_type=jnp.float32) * SCALE
        p = jnp.where(mask, jnp.exp(sc), 0.0).astype(jnp.bfloat16)
        o_h = jnp.dot(p, v3_ref[hd],
                      preferred_element_type=jnp.float32)
        l8 = jnp.concatenate(
            [lax.dot_general(ones_row, p[a * DH:(a + 1) * DH, :],
                             (((1,), (1,)), ((), ())),
                             preferred_element_type=jnp.float32)
             for a in range(HQ)], axis=0)
        return o_h, l8

    def p1_send(idx, hd):
        return pltpu.make_async_remote_copy(
            src_ref=stage_ref.at[hd],
            dst_ref=comm_ref.at[pos],
            send_sem=send1.at[idx],
            recv_sem=recv1.at[pos],
            device_id=(hd,),
            device_id_type=pl.DeviceIdType.MESH,
        )

    for idx in range(N_DEV - 1):
        hd = lax.rem(pos + 1 + idx, N_DEV)
        o_h, l8 = head_partial(hd)
        stage_ref[hd] = jnp.concatenate(
            [o_h.astype(jnp.bfloat16), l8.astype(jnp.bfloat16)], axis=0)
        p1_send(idx, hd).start()
    o_own, l8_own = head_partial(pos)

    acc = jnp.concatenate([o_own, l8_own], axis=0)
    for k in range(N_DEV - 1):
        src = lax.rem(pos + (N_DEV - 1) - k, N_DEV)
        pltpu.make_async_remote_copy(
            src_ref=comm_ref.at[src], dst_ref=comm_ref.at[src],
            send_sem=send1.at[0], recv_sem=recv1.at[src],
            device_id=(pos,), device_id_type=pl.DeviceIdType.MESH,
        ).wait_recv()
        acc = acc + comm_ref[src].astype(jnp.float32)

    t = lax.dot_general(ieye, acc[SQ:, :].astype(jnp.bfloat16),
                        (((1,), (1,)), ((), ())),
                        preferred_element_type=jnp.float32)
    l_col = jnp.concatenate([t[:, a:a + 1] for a in range(HQ)],
                            axis=0)
    ctx_mine = (acc[:SQ, :] / l_col).astype(jnp.bfloat16)
    stage_ref[pos, :SQ] = ctx_mine

    def p2_send(off):
        return pltpu.make_async_remote_copy(
            src_ref=stage_ref.at[pos, pl.ds(0, SQ)],
            dst_ref=ctx_ref.at[pos],
            send_sem=send2.at[off],
            recv_sem=recv2.at[pos],
            device_id=(lax.rem(pos + off, N_DEV),),
            device_id_type=pl.DeviceIdType.MESH,
        )

    for off in range(1, N_DEV):
        p2_send(off).start()

    out = jnp.dot(ctx_mine,
                  wo_ref[pl.ds(pos * DH, DH), :].astype(jnp.bfloat16),
                  preferred_element_type=jnp.float32)
    for k in range(N_DEV - 1):
        src = lax.rem(pos + (N_DEV - 1) - k, N_DEV)
        pltpu.make_async_remote_copy(
            src_ref=ctx_ref.at[src], dst_ref=ctx_ref.at[src],
            send_sem=send2.at[0], recv_sem=recv2.at[src],
            device_id=(pos,), device_id_type=pl.DeviceIdType.MESH,
        ).wait_recv()
        out = out + jnp.dot(
            ctx_ref[src],
            wo_ref[pl.ds(src * DH, DH), :].astype(jnp.bfloat16),
            preferred_element_type=jnp.float32)
    out_ref[0] = out

    for idx in range(N_DEV - 1):
        p1_send(idx, lax.rem(pos + 1 + idx, N_DEV)).wait_send()
    for off in range(1, N_DEV):
        p2_send(off).wait_send()


def kernel(x, Wq, K_ext, V_ext, Wo):
    return pl.pallas_call(
        _body,
        out_shape=jax.ShapeDtypeStruct((1, SQ, D), jnp.float32),
        in_specs=[pl.BlockSpec(memory_space=pltpu.VMEM)] * 5,
        out_specs=pl.BlockSpec(memory_space=pltpu.VMEM),
        scratch_shapes=[
            pltpu.VMEM((HQ, SQ, DH), jnp.bfloat16),
            pltpu.VMEM((HQ, SKV_LOCAL, DH), jnp.bfloat16),
            pltpu.VMEM((HQ, SKV_LOCAL, DH), jnp.bfloat16),
            pltpu.VMEM((HQ, PACK, DH), jnp.bfloat16),
            pltpu.VMEM((N_DEV, PACK, DH), jnp.bfloat16),
            pltpu.VMEM((N_DEV, SQ, DH), jnp.bfloat16),
            pltpu.SemaphoreType.DMA((N_DEV,)),
            pltpu.SemaphoreType.DMA((N_DEV,)),
            pltpu.SemaphoreType.DMA((N_DEV,)),
            pltpu.SemaphoreType.DMA((N_DEV,)),
        ],
        compiler_params=pltpu.CompilerParams(
            collective_id=0,
            vmem_limit_bytes=100 * 1024 * 1024,
        ),
    )(x[0], Wq, K_ext[0].reshape(SKV_LOCAL, D), V_ext[0].reshape(SKV_LOCAL, D),
      Wo)
